# Initial kernel scaffold; baseline (speedup 1.0000x reference)
import jax
import jax.numpy as jnp
from jax.experimental import pallas as pl


def kernel(input_tensor, juncs_pred_hawp, device):
    raise NotImplementedError("write your pallas kernel here")


# speedup vs baseline: 3.9803x; 3.9803x over previous
import functools

import jax
import jax.numpy as jnp
from jax import lax
from jax.experimental import pallas as pl
from jax.experimental.pallas import tpu as pltpu
from jax.experimental.pallas import tpu_sc as plsc

_B = 2
_H = 512
_W = 512
_NJ = 512
_K = 512
_C = 128
_HW = _H * _W
_THRC = 0.99


def _shift(a, k, axis, left):
    ninf = jnp.float32(-jnp.inf)
    if axis == 1:
        pad = jnp.full((a.shape[0], k), ninf, a.dtype)
        if left:
            return jnp.concatenate([a[:, k:], pad], axis=1)
        return jnp.concatenate([pad, a[:, : a.shape[1] - k]], axis=1)
    pad = jnp.full((k, a.shape[1]), ninf, a.dtype)
    if left:
        return jnp.concatenate([a[k:, :], pad], axis=0)
    return jnp.concatenate([pad, a[: a.shape[0] - k, :]], axis=0)


def _shift_back_clamped(a, k, axis):
    if axis == 1:
        edge = jnp.broadcast_to(a[:, 0:1], (a.shape[0], k))
        return jnp.concatenate([edge, a[:, : a.shape[1] - k]], axis=1)
    edge = jnp.broadcast_to(a[0:1, :], (k, a.shape[1]))
    return jnp.concatenate([edge, a[: a.shape[0] - k, :]], axis=0)


def _win15(a, axis):
    g = jnp.maximum(a, _shift(a, 1, axis, True))
    g = jnp.maximum(g, _shift(g, 2, axis, True))
    g = jnp.maximum(g, _shift(g, 4, axis, True))
    return jnp.maximum(g, _shift_back_clamped(g, 7, axis))


def _nms_body(heat_ref, juncs_ref, sc_ref, x_ref, y_ref, idx_ref):
    b = pl.program_id(0)
    h = heat_ref[0]
    pooled = _win15(_win15(h, 1), 0)
    rows = lax.broadcasted_iota(jnp.int32, (_H, _W), 0)
    cols = lax.broadcasted_iota(jnp.int32, (_H, _W), 1)
    flat = rows * _W + cols
    mask = (h == pooled) & (h > _THRC)
    sp = jnp.where(mask, h, -1.0 - flat.astype(jnp.float32))

    sp3 = sp.reshape(_H // 8, 8, _W)
    vy = jnp.max(sp3, axis=1)
    fl3 = flat.reshape(_H // 8, 8, _W)
    iy = jnp.min(jnp.where(sp3 == vy[:, None, :], fl3, jnp.int32(2**30)),
                 axis=1)
    vyt = vy.T
    iyt = iy.T
    v3 = vyt.reshape(_W // 8, 8, _H // 8)
    vt = jnp.max(v3, axis=1)
    i3 = iyt.reshape(_W // 8, 8, _H // 8)
    it = jnp.min(jnp.where(v3 == vt[:, None, :], i3, jnp.int32(2**30)),
                 axis=1)

    V3 = vt[:, None, :]
    If64 = it.astype(jnp.float32)
    I3 = If64[:, None, :]
    VT = vt.T
    IT = If64.T
    rank = jnp.zeros((64, 1, 64), jnp.float32)
    for jb in range(64):
        col = VT[:, jb:jb + 1][None]
        coli = IT[:, jb:jb + 1][None]
        beats = (col > V3) | ((col == V3) & (coli < I3))
        rank = rank + jnp.sum(beats.astype(jnp.float32), axis=1,
                              keepdims=True)

    ranki = rank.astype(jnp.int32)
    kio = lax.broadcasted_iota(jnp.int32, (1, 128, 1), 1)
    svs = []
    sis = []
    for kc in range(4):
        sel = ranki == (kio + 128 * kc)
        sv_c = jnp.sum(jnp.sum(jnp.where(sel, V3, 0.0), axis=2), axis=0,
                       keepdims=True).T
        si_c = jnp.sum(jnp.sum(jnp.where(sel, I3, 0.0), axis=2), axis=0,
                       keepdims=True).T
        svs.append(sv_c)
        sis.append(si_c)
    sv = jnp.concatenate(svs, axis=0)
    si = jnp.concatenate(sis, axis=0).astype(jnp.int32)

    sc_ref[0] = (sv / jnp.max(sv)).T
    yv = si // _W
    xv = si - yv * _W
    x_ref[0] = xv.T
    y_ref[0] = yv.T

    jx = juncs_ref[0][:, 0:1]
    jy = juncs_ref[0][:, 1:2]
    pj = jy * _W + jx
    pall = jnp.concatenate([pj, si], axis=0)
    cio = lax.broadcasted_iota(jnp.int32, (1, _C), 1)
    idx_ref[0] = pall + (cio + 1) * _HW + b * (129 * _HW)


def _nms_topk(heat, juncs):
    return pl.pallas_call(
        _nms_body,
        grid=(_B,),
        in_specs=[
            pl.BlockSpec((1, _H, _W), lambda b: (b, 0, 0)),
            pl.BlockSpec((1, _NJ, 2), lambda b: (b, 0, 0)),
        ],
        out_specs=[
            pl.BlockSpec((1, 1, _K), lambda b: (b, 0, 0)),
            pl.BlockSpec((1, 1, _K), lambda b: (b, 0, 0)),
            pl.BlockSpec((1, 1, _K), lambda b: (b, 0, 0)),
            pl.BlockSpec((1, _NJ + _K, _C), lambda b: (b, 0, 0)),
        ],
        out_shape=[
            jax.ShapeDtypeStruct((_B, 1, _K), jnp.float32),
            jax.ShapeDtypeStruct((_B, 1, _K), jnp.int32),
            jax.ShapeDtypeStruct((_B, 1, _K), jnp.int32),
            jax.ShapeDtypeStruct((_B, _NJ + _K, _C), jnp.int32),
        ],
    )(heat, juncs)


_NW = 32
_NG = _B * (_NJ + _K) * _C
_CHUNK = _NG // _NW


def _sc_gather(flat_src, idx_flat):
    mesh = plsc.VectorSubcoreMesh(core_axis_name="c", subcore_axis_name="s")

    @functools.partial(
        pl.kernel,
        mesh=mesh,
        out_type=jax.ShapeDtypeStruct((_NG,), jnp.float32),
        scratch_types=[
            pltpu.VMEM((_CHUNK,), jnp.int32),
            pltpu.VMEM((_CHUNK,), jnp.float32),
            pltpu.SemaphoreType.DMA,
        ],
    )
    def gather_k(src_hbm, idx_hbm, out_hbm, idx_v, val_v, sem):
        wid = lax.axis_index("s") * 2 + lax.axis_index("c")
        base = wid * _CHUNK
        pltpu.sync_copy(idx_hbm.at[pl.ds(base, _CHUNK)], idx_v)
        pltpu.async_copy(src_hbm.at[idx_v], val_v, sem).wait()
        pltpu.sync_copy(val_v, out_hbm.at[pl.ds(base, _CHUNK)])

    return gather_k(flat_src, idx_flat)


def _norm_body(d_ref, o_ref):
    d = d_ref[0]
    ss = jnp.sum(d * d, axis=1, keepdims=True)
    n = jnp.sqrt(ss)
    o_ref[0] = d / jnp.maximum(n, jnp.float32(1e-12))


def _normalize(draw):
    return pl.pallas_call(
        _norm_body,
        grid=(_B,),
        in_specs=[pl.BlockSpec((1, _NJ + _K, _C), lambda b: (b, 0, 0))],
        out_specs=pl.BlockSpec((1, _NJ + _K, _C), lambda b: (b, 0, 0)),
        out_shape=jax.ShapeDtypeStruct((_B, _NJ + _K, _C), jnp.float32),
    )(draw)


def kernel(input_tensor, juncs_pred_hawp, device):
    heat = input_tensor[:, 0]
    juncs = juncs_pred_hawp.astype(jnp.int32)
    scores_kp, xk, yk, gidx = _nms_topk(heat, juncs)
    vals = _sc_gather(input_tensor.reshape(-1), gidx.reshape(-1))
    descriptors = _normalize(vals.reshape(_B, _NJ + _K, _C))
    kps = jnp.stack([xk[:, 0, :], yk[:, 0, :]], axis=-1)
    keypoints_final = jnp.concatenate([juncs, kps], axis=1)
    scores = jnp.concatenate(
        [jnp.ones((_B, _NJ), jnp.float32), scores_kp[:, 0, :]], axis=-1)
    return (keypoints_final, descriptors, scores)
